# baseline (speedup 1.0000x reference)
import functools

import jax
import jax.numpy as jnp
from jax import lax
from jax.experimental import pallas as pl
from jax.experimental.pallas import tpu as pltpu
from jax.experimental.pallas import tpu_sc as plsc

_VOCAB_PAD = 30720
_BLK = 1024
_SEQ = 200
_BATCH = 4096
_OUT = 2
_NC, _NS, _L = 2, 16, 16
_NW = _NC * _NS
_BPW = _BATCH // _NW
_G = _BPW // _L


def _project_body(e_ref, w_ref, b_ref, t_ref):
    t_ref[...] = (
        jnp.dot(e_ref[...], w_ref[...], preferred_element_type=jnp.float32)
        + b_ref[...]
    )


def _project(emb_table, W, b):
    d = emb_table.shape[1]
    return pl.pallas_call(
        _project_body,
        grid=(_VOCAB_PAD // _BLK,),
        in_specs=[
            pl.BlockSpec((_BLK, d), lambda i: (i, 0)),
            pl.BlockSpec((d, _OUT), lambda i: (0, 0)),
            pl.BlockSpec((1, _OUT), lambda i: (0, 0)),
        ],
        out_specs=pl.BlockSpec((_BLK, _OUT), lambda i: (i, 0)),
        out_shape=jax.ShapeDtypeStruct((_VOCAB_PAD, _OUT), jnp.float32),
    )(emb_table, W, b.reshape(1, _OUT))


def _sc_body(t_hbm, x_hbm, out_hbm, t_v, x_v, o_v):
    wid = lax.axis_index("s") * _NC + lax.axis_index("c")
    pltpu.sync_copy(t_hbm, t_v)
    pltpu.sync_copy(x_hbm.at[pl.ds(wid * _BPW * _SEQ, _BPW * _SEQ)], x_v)
    inv_l = jnp.float32(1.0 / _SEQ)
    lanes = lax.iota(jnp.int32, _L)
    z = jnp.zeros((_L,), jnp.float32)
    for g in range(_G):
        rows = g * _L + lanes
        base = rows * _SEQ

        @plsc.parallel_loop(0, _SEQ, step=1, unroll=8, carry=(z, z))
        def body(l, accs, _base=base):
            a0, a1 = accs
            idx = plsc.load_gather(x_v, [_base + l])
            idx2 = idx + idx
            v0 = plsc.load_gather(t_v, [idx2])
            v1 = plsc.load_gather(t_v, [idx2 + 1])
            return a0 + v0, a1 + v1

        a0, a1 = body
        oaddr = rows * _OUT
        plsc.store_scatter(o_v, [oaddr], a0 * inv_l)
        plsc.store_scatter(o_v, [oaddr + 1], a1 * inv_l)
    pltpu.sync_copy(o_v, out_hbm.at[pl.ds(wid * _BPW * _OUT, _BPW * _OUT)])


_sc_pool = functools.partial(
    pl.kernel,
    out_type=jax.ShapeDtypeStruct((_BATCH * _OUT,), jnp.float32),
    mesh=plsc.VectorSubcoreMesh(
        core_axis_name="c", subcore_axis_name="s",
        num_cores=_NC, num_subcores=_NS,
    ),
    scratch_types=[
        pltpu.VMEM((_VOCAB_PAD * _OUT,), jnp.float32),
        pltpu.VMEM((_BPW * _SEQ,), jnp.int32),
        pltpu.VMEM((_BPW * _OUT,), jnp.float32),
    ],
    compiler_params=pltpu.CompilerParams(needs_layout_passes=False),
)(_sc_body)


def kernel(x, attention_mask, emb_table, W, b):
    t = _project(emb_table, W, b).reshape(-1)
    out = _sc_pool(t, x.reshape(-1))
    return out.reshape(_BATCH, _OUT)


# speedup vs baseline: 1.1522x; 1.1522x over previous
import functools

import jax
import jax.numpy as jnp
from jax import lax
from jax.experimental import pallas as pl
from jax.experimental.pallas import tpu as pltpu
from jax.experimental.pallas import tpu_sc as plsc

_VOCAB_PAD = 30720
_BLK = 1024
_SEQ = 200
_BATCH = 4096
_OUT = 2
_NC, _NS, _L = 2, 16, 16
_NW = _NC * _NS
_BPW = _BATCH // _NW
_G = _BPW // _L


def _project_body(e_ref, w_ref, b_ref, t_ref):
    t_ref[...] = (
        jnp.dot(e_ref[...], w_ref[...], preferred_element_type=jnp.float32)
        + b_ref[...]
    )


def _project(emb_table, W, b):
    d = emb_table.shape[1]
    return pl.pallas_call(
        _project_body,
        grid=(_VOCAB_PAD // _BLK,),
        in_specs=[
            pl.BlockSpec((_BLK, d), lambda i: (i, 0)),
            pl.BlockSpec((d, _OUT), lambda i: (0, 0)),
            pl.BlockSpec((1, _OUT), lambda i: (0, 0)),
        ],
        out_specs=pl.BlockSpec((_BLK, _OUT), lambda i: (i, 0)),
        out_shape=jax.ShapeDtypeStruct((_VOCAB_PAD, _OUT), jnp.float32),
    )(emb_table, W, b.reshape(1, _OUT))


def _sc_body(t_hbm, x_hbm, out_hbm, t_v, x_v, o_v):
    wid = lax.axis_index("s") * _NC + lax.axis_index("c")
    pltpu.sync_copy(t_hbm, t_v)
    pltpu.sync_copy(x_hbm.at[wid], x_v)
    inv_l = jnp.float32(1.0 / _SEQ)
    z = jnp.zeros((_L,), jnp.float32)
    for g in range(_G):
        @plsc.parallel_loop(0, _SEQ, step=1, unroll=8, carry=(z, z))
        def body(l, accs, _g=g):
            a0, a1 = accs
            idx2 = x_v[l, pl.ds(_g * _L, _L)] * 2
            v0 = plsc.load_gather(t_v, [idx2])
            v1 = plsc.load_gather(t_v, [idx2 + 1])
            return a0 + v0, a1 + v1

        a0, a1 = body
        o_v[0, pl.ds(g * _L, _L)] = a0 * inv_l
        o_v[1, pl.ds(g * _L, _L)] = a1 * inv_l
    pltpu.sync_copy(o_v, out_hbm.at[wid])


_sc_pool = functools.partial(
    pl.kernel,
    out_type=jax.ShapeDtypeStruct((_NW, _OUT, _BPW), jnp.float32),
    mesh=plsc.VectorSubcoreMesh(
        core_axis_name="c", subcore_axis_name="s",
        num_cores=_NC, num_subcores=_NS,
    ),
    scratch_types=[
        pltpu.VMEM((_VOCAB_PAD * _OUT,), jnp.float32),
        pltpu.VMEM((_SEQ, _BPW), jnp.int32),
        pltpu.VMEM((_OUT, _BPW), jnp.float32),
    ],
    compiler_params=pltpu.CompilerParams(needs_layout_passes=False),
)(_sc_body)


def kernel(x, attention_mask, emb_table, W, b):
    t = _project(emb_table, W, b).reshape(-1)
    x2 = x.reshape(_NW, _BPW, _SEQ).transpose(0, 2, 1)
    out = _sc_pool(t, x2)
    return out.transpose(0, 2, 1).reshape(_BATCH, _OUT)


# speedup vs baseline: 1.2570x; 1.0909x over previous
import functools

import jax
import jax.numpy as jnp
from jax import lax
from jax.experimental import pallas as pl
from jax.experimental.pallas import tpu as pltpu
from jax.experimental.pallas import tpu_sc as plsc

_VOCAB_PAD = 30720
_BLK = 2048
_SEQ = 200
_BATCH = 4096
_OUT = 2
_NC, _NS, _L = 2, 16, 16
_NW = _NC * _NS
_BPW = _BATCH // _NW
_G = _BPW // _L


def _project_body(e_ref, w_ref, b_ref, t_ref):
    t_ref[...] = (
        jnp.dot(e_ref[...], w_ref[...], preferred_element_type=jnp.float32)
        + b_ref[...]
    )


def _project(emb_table, W, b):
    d = emb_table.shape[1]
    return pl.pallas_call(
        _project_body,
        grid=(_VOCAB_PAD // _BLK,),
        in_specs=[
            pl.BlockSpec((_BLK, d), lambda i: (i, 0)),
            pl.BlockSpec((d, _OUT), lambda i: (0, 0)),
            pl.BlockSpec((1, _OUT), lambda i: (0, 0)),
        ],
        out_specs=pl.BlockSpec((_BLK, _OUT), lambda i: (i, 0)),
        out_shape=jax.ShapeDtypeStruct((_VOCAB_PAD, _OUT), jnp.float32),
    )(emb_table, W, b.reshape(1, _OUT))


def _sc_body(t_hbm, x_hbm, out_hbm, t_v, x_v, o_v):
    wid = lax.axis_index("s") * _NC + lax.axis_index("c")
    pltpu.sync_copy(t_hbm, t_v)
    pltpu.sync_copy(x_hbm.at[wid], x_v)
    inv_l = jnp.float32(1.0 / _SEQ)
    z = jnp.zeros((_L,), jnp.float32)
    for g in range(_G):
        @plsc.parallel_loop(0, _SEQ, step=2, unroll=4, carry=(z, z, z, z))
        def body(l, accs, _g=g):
            a0, a1, b0, b1 = accs
            idx2 = x_v[l, pl.ds(_g * _L, _L)] * 2
            jdx2 = x_v[l + 1, pl.ds(_g * _L, _L)] * 2
            a0 = a0 + plsc.load_gather(t_v, [idx2])
            a1 = a1 + plsc.load_gather(t_v, [idx2 + 1])
            b0 = b0 + plsc.load_gather(t_v, [jdx2])
            b1 = b1 + plsc.load_gather(t_v, [jdx2 + 1])
            return a0, a1, b0, b1

        a0, a1, b0, b1 = body
        o_v[0, pl.ds(g * _L, _L)] = (a0 + b0) * inv_l
        o_v[1, pl.ds(g * _L, _L)] = (a1 + b1) * inv_l
    pltpu.sync_copy(o_v, out_hbm.at[wid])


_sc_pool = functools.partial(
    pl.kernel,
    out_type=jax.ShapeDtypeStruct((_NW, _OUT, _BPW), jnp.float32),
    mesh=plsc.VectorSubcoreMesh(
        core_axis_name="c", subcore_axis_name="s",
        num_cores=_NC, num_subcores=_NS,
    ),
    scratch_types=[
        pltpu.VMEM((_VOCAB_PAD * _OUT,), jnp.float32),
        pltpu.VMEM((_SEQ, _BPW), jnp.int32),
        pltpu.VMEM((_OUT, _BPW), jnp.float32),
    ],
    compiler_params=pltpu.CompilerParams(needs_layout_passes=False),
)(_sc_body)


def kernel(x, attention_mask, emb_table, W, b):
    t = _project(emb_table, W, b).reshape(-1)
    x2 = x.reshape(_NW, _BPW, _SEQ).transpose(0, 2, 1)
    out = _sc_pool(t, x2)
    return out.transpose(0, 2, 1).reshape(_BATCH, _OUT)
